# baseline (speedup 1.0000x reference)
import functools

import jax
import jax.numpy as jnp
from jax import lax
from jax.experimental import pallas as pl
from jax.experimental.pallas import tpu as pltpu
from jax.experimental.pallas import tpu_sc as plsc

_B, _C, _K1, _D = 4096, 10, 6, 64
_NW = 32
_BPW = _B // _NW
_HALF = _BPW // 2
_LG = 16
_ND = _D // _LG
_ROWS = _B * _K1 * _LG // 128


def _sc_scores(u_idx, w_idx, u_weight, w_weight):
    mesh = plsc.VectorSubcoreMesh(core_axis_name="c", subcore_axis_name="s")

    @functools.partial(
        pl.kernel,
        out_type=jax.ShapeDtypeStruct((_B * _K1, _LG), jnp.float32),
        mesh=mesh,
        scratch_types=[
            pltpu.VMEM((_BPW * _C,), jnp.int32),
            pltpu.VMEM((_BPW * _K1,), jnp.int32),
            pltpu.VMEM((_HALF * _C, _D), jnp.float32),
            pltpu.VMEM((_HALF * _K1, _D), jnp.float32),
            pltpu.VMEM((_HALF * _K1, _LG), jnp.float32),
            pltpu.SemaphoreType.DMA,
        ],
        compiler_params=pltpu.CompilerParams(use_tc_tiling_on_sc=False),
    )
    def body(u_idx_hbm, w_idx_hbm, uw_hbm, ww_hbm, out_hbm,
             u_idx_v, w_idx_v, u_rows, w_rows, out_v, sem):
        wid = lax.axis_index("s") * 2 + lax.axis_index("c")
        base = wid * _BPW
        pltpu.sync_copy(u_idx_hbm.at[pl.ds(base * _C, _BPW * _C)], u_idx_v)
        pltpu.sync_copy(w_idx_hbm.at[pl.ds(base * _K1, _BPW * _K1)], w_idx_v)

        for half in range(_BPW // _HALF):
            off = half * _HALF
            copies = []
            for g in range(_HALF * _C // 128):
                copies.append(pltpu.async_copy(
                    uw_hbm.at[u_idx_v.at[pl.ds(off * _C + g * 128, 128)]],
                    u_rows.at[pl.ds(g * 128, 128)], sem))
            for g in range(_HALF * _K1 // 128):
                copies.append(pltpu.async_copy(
                    ww_hbm.at[w_idx_v.at[pl.ds(off * _K1 + g * 128, 128)]],
                    w_rows.at[pl.ds(g * 128, 128)], sem))
            for cp in copies:
                cp.wait()

            def elem(e, carry):
                accs = []
                for d in range(_ND):
                    a = u_rows[e * _C, pl.ds(d * _LG, _LG)]
                    for c in range(1, _C):
                        a = a + u_rows[e * _C + c, pl.ds(d * _LG, _LG)]
                    accs.append(a)
                for t in range(_K1):
                    p = accs[0] * w_rows[e * _K1 + t, pl.ds(0, _LG)]
                    for d in range(1, _ND):
                        p = p + accs[d] * w_rows[e * _K1 + t,
                                                 pl.ds(d * _LG, _LG)]
                    if t > 0:
                        p = -p
                    out_v[e * _K1 + t, pl.ds(0, _LG)] = p
                return carry

            lax.fori_loop(0, _HALF, elem, 0)
            pltpu.sync_copy(
                out_v, out_hbm.at[pl.ds((base + off) * _K1, _HALF * _K1)])

    return body(u_idx, w_idx, u_weight, w_weight)


def _tc_loss_body(x_ref, o_ref):
    x = x_ref[...]
    col = lax.broadcasted_iota(jnp.int32, (128, 8), 0) // _LG
    grp = lax.broadcasted_iota(jnp.int32, (128, 8), 1)
    sel = (col == grp).astype(jnp.float32)
    s = jnp.dot(x, sel, preferred_element_type=jnp.float32)
    ls = jnp.minimum(s, 0.0) - jnp.log1p(jnp.exp(-jnp.abs(s)))
    o_ref[0, 0] = -jnp.sum(ls)


def _tc_loss(lanes_flat):
    return pl.pallas_call(
        _tc_loss_body,
        out_shape=jax.ShapeDtypeStruct((1, 1), jnp.float32),
        in_specs=[pl.BlockSpec(memory_space=pltpu.VMEM)],
        out_specs=pl.BlockSpec(memory_space=pltpu.SMEM),
    )(lanes_flat)


def kernel(pos_u, pos_w, neg_w, u_weight, w_weight):
    u_idx = pos_u.astype(jnp.int32).reshape(_B * _C)
    w_idx = jnp.concatenate(
        [pos_w.astype(jnp.int32)[:, None], neg_w.astype(jnp.int32)],
        axis=1).reshape(_B * _K1)
    lanes = _sc_scores(u_idx, w_idx, u_weight, w_weight)
    return _tc_loss(lanes.reshape(_ROWS, 128))[0, 0]


# speedup vs baseline: 1.0511x; 1.0511x over previous
import functools

import jax
import jax.numpy as jnp
from jax import lax
from jax.experimental import pallas as pl
from jax.experimental.pallas import tpu as pltpu
from jax.experimental.pallas import tpu_sc as plsc

_B, _C, _K1, _D = 4096, 10, 6, 64
_NW = 32
_BPW = _B // _NW
_HALF = _BPW // 2
_LG = 16
_ND = _D // _LG
_ROWS = _B * _K1 * _LG // 128


_CH = 32
_NROUND = _BPW // _CH
_DP = 128


def _streams(total):
    out, off = [], 0
    while off < total:
        n = min(128, total - off)
        out.append((off, n))
        off += n
    return out


def _sc_scores(u_idx, w_idx, u_weight, w_weight):
    mesh = plsc.VectorSubcoreMesh(core_axis_name="c", subcore_axis_name="s")

    @functools.partial(
        pl.kernel,
        out_type=jax.ShapeDtypeStruct((_B * _K1, _LG), jnp.float32),
        mesh=mesh,
        scratch_types=[
            pltpu.VMEM((_BPW * _C,), jnp.int32),
            pltpu.VMEM((_BPW * _K1,), jnp.int32),
            pltpu.VMEM((_CH * _C, _DP), jnp.float32),
            pltpu.VMEM((_CH * _K1, _DP), jnp.float32),
            pltpu.VMEM((_CH * _K1, _LG), jnp.float32),
            pltpu.SemaphoreType.DMA,
        ],
    )
    def body(u_idx_hbm, w_idx_hbm, uw_hbm, ww_hbm, out_hbm,
             u_idx_v, w_idx_v, u_rows, w_rows, out_v, sem):
        wid = lax.axis_index("s") * 2 + lax.axis_index("c")
        base = wid * _BPW
        pltpu.sync_copy(u_idx_hbm.at[pl.ds(base * _C, _BPW * _C)], u_idx_v)
        pltpu.sync_copy(w_idx_hbm.at[pl.ds(base * _K1, _BPW * _K1)], w_idx_v)

        for rnd in range(_NROUND):
            off = rnd * _CH
            copies = []
            for (so, sn) in _streams(_CH * _C):
                copies.append(pltpu.async_copy(
                    uw_hbm.at[u_idx_v.at[pl.ds(off * _C + so, sn)]],
                    u_rows.at[pl.ds(so, sn)], sem))
            for (so, sn) in _streams(_CH * _K1):
                copies.append(pltpu.async_copy(
                    ww_hbm.at[w_idx_v.at[pl.ds(off * _K1 + so, sn)]],
                    w_rows.at[pl.ds(so, sn)], sem))
            for cp in copies:
                cp.wait()

            def elem(e, carry):
                accs = []
                for d in range(_ND):
                    a = u_rows[e * _C, pl.ds(d * _LG, _LG)]
                    for c in range(1, _C):
                        a = a + u_rows[e * _C + c, pl.ds(d * _LG, _LG)]
                    accs.append(a)
                for t in range(_K1):
                    p = accs[0] * w_rows[e * _K1 + t, pl.ds(0, _LG)]
                    for d in range(1, _ND):
                        p = p + accs[d] * w_rows[e * _K1 + t,
                                                 pl.ds(d * _LG, _LG)]
                    if t > 0:
                        p = -p
                    out_v[e * _K1 + t, pl.ds(0, _LG)] = p
                return carry

            lax.fori_loop(0, _CH, elem, 0)
            pltpu.sync_copy(
                out_v, out_hbm.at[pl.ds((base + off) * _K1, _CH * _K1)])

    return body(u_idx, w_idx, u_weight, w_weight)


def _tc_loss_body(x_ref, o_ref):
    x = x_ref[...]
    col = lax.broadcasted_iota(jnp.int32, (128, 8), 0) // _LG
    grp = lax.broadcasted_iota(jnp.int32, (128, 8), 1)
    sel = (col == grp).astype(jnp.float32)
    s = jnp.dot(x, sel, preferred_element_type=jnp.float32)
    ls = jnp.minimum(s, 0.0) - jnp.log1p(jnp.exp(-jnp.abs(s)))
    o_ref[0, 0] = -jnp.sum(ls)


def _tc_loss(lanes_flat):
    return pl.pallas_call(
        _tc_loss_body,
        out_shape=jax.ShapeDtypeStruct((1, 1), jnp.float32),
        in_specs=[pl.BlockSpec(memory_space=pltpu.VMEM)],
        out_specs=pl.BlockSpec(memory_space=pltpu.SMEM),
    )(lanes_flat)


def kernel(pos_u, pos_w, neg_w, u_weight, w_weight):
    u_idx = pos_u.astype(jnp.int32).reshape(_B * _C)
    w_idx = jnp.concatenate(
        [pos_w.astype(jnp.int32)[:, None], neg_w.astype(jnp.int32)],
        axis=1).reshape(_B * _K1)
    u128 = jnp.pad(u_weight, ((0, 0), (0, _DP - _D)))
    w128 = jnp.pad(w_weight, ((0, 0), (0, _DP - _D)))
    lanes = _sc_scores(u_idx, w_idx, u128, w128)
    return _tc_loss(lanes.reshape(_ROWS, 128))[0, 0]
